# Initial kernel scaffold; baseline (speedup 1.0000x reference)
import jax
import jax.numpy as jnp
from jax.experimental import pallas as pl


def kernel(points, x, W, gamma, beta):
    raise NotImplementedError("write your pallas kernel here")


# speedup vs baseline: 13.2543x; 13.2543x over previous
import functools

import jax
import jax.numpy as jnp
from jax import lax
from jax.experimental import pallas as pl
from jax.experimental.pallas import tpu as pltpu
from jax.experimental.pallas import tpu_sc as plsc

KNB = 16
ALPHA = 0.2
EPS = 1e-5
NEG = -3.0e38

NC, NS = 2, 16
NW = NC * NS

B, F, D, N = 4, 3, 64, 4096
C = 64
RBLK = 128
NBLK = N // RBLK
BN = B * N
ROWS_W = BN // NW


def _knn_body(pts_all_ref, pts_row_ref, xt_ref, w1t_ref, w2t_ref,
              idx_ref, y1_ref, y2_ref):
    b = pl.program_id(0)
    pall = pts_all_ref[0]
    prow = pts_row_ref[0]
    xx = jnp.sum(pall * pall, axis=0)
    xxr = jnp.sum(prow * prow, axis=0)
    inner = lax.dot_general(prow, pall, (((0,), (0,)), ((), ())))
    d = 2.0 * inner - xxr[:, None] - xx[None, :]

    iota = lax.broadcasted_iota(jnp.int32, (RBLK, N), 1)
    kiota = lax.broadcasted_iota(jnp.int32, (RBLK, KNB), 1)
    midx = jnp.zeros((RBLK, KNB), dtype=jnp.int32)
    for j in range(KNB):
        m = jnp.max(d, axis=1)
        sel = d == m[:, None]
        ij = jnp.min(jnp.where(sel, iota, N), axis=1)
        midx = jnp.where(kiota == j, ij[:, None], midx)
        d = jnp.where(iota == ij[:, None], NEG, d)
    idx_ref[0] = midx + b * N

    xrow = xt_ref[0]
    y1_ref[0] = jnp.dot(xrow, w1t_ref[...],
                        preferred_element_type=jnp.float32)
    y2_ref[0] = jnp.dot(xrow, w2t_ref[...],
                        preferred_element_type=jnp.float32)


def _run_knn(points, xt, w1t, w2t):
    return pl.pallas_call(
        _knn_body,
        grid=(B, NBLK),
        in_specs=[
            pl.BlockSpec((1, F, N), lambda b, i: (b, 0, 0)),
            pl.BlockSpec((1, F, RBLK), lambda b, i: (b, 0, i)),
            pl.BlockSpec((1, RBLK, D), lambda b, i: (b, i, 0)),
            pl.BlockSpec((D, C), lambda b, i: (0, 0)),
            pl.BlockSpec((D, C), lambda b, i: (0, 0)),
        ],
        out_specs=[
            pl.BlockSpec((1, RBLK, KNB), lambda b, i: (b, i, 0)),
            pl.BlockSpec((1, RBLK, C), lambda b, i: (b, i, 0)),
            pl.BlockSpec((1, RBLK, C), lambda b, i: (b, i, 0)),
        ],
        out_shape=[
            jax.ShapeDtypeStruct((B, N, KNB), jnp.int32),
            jax.ShapeDtypeStruct((B, N, C), jnp.float32),
            jax.ShapeDtypeStruct((B, N, C), jnp.float32),
        ],
    )(points, points, xt, w1t, w2t)


G1 = 8
NB1 = ROWS_W // G1
E1 = G1 * KNB


def _pass1_body(y1_hbm, idx_hbm, y2_hbm, part_hbm,
                idxl, y2l, buf0, buf1, acc, sem0, sem1):
    wid = lax.axis_index("s") * NC + lax.axis_index("c")

    pltpu.sync_copy(idx_hbm.at[pl.ds(wid * NB1, NB1)], idxl)
    pltpu.sync_copy(y2_hbm.at[pl.ds(wid * ROWS_W * C, ROWS_W * C)], y2l)

    zero = jnp.zeros((16,), jnp.float32)
    for t in range(12):
        acc[pl.ds(t * 16, 16)] = zero

    def start(g, buf, sem):
        pltpu.async_copy(y1_hbm.at[idxl.at[g]], buf, sem)

    def wait(g, buf, sem):
        pltpu.make_async_copy(y1_hbm.at[idxl.at[g]], buf, sem).wait()

    def compute(g, buf):
        s1a = [None] * 4
        sqa = [None] * 4
        cra = [None] * 4
        for j in range(G1):
            s1 = [None] * 4
            sq = [None] * 4
            for t in range(KNB):
                e = j * KNB + t
                for c in range(4):
                    v = buf[e, pl.ds(c * 16, 16)]
                    if t == 0:
                        s1[c] = v
                        sq[c] = v * v
                    else:
                        s1[c] = s1[c] + v
                        sq[c] = sq[c] + v * v
            rbase = (g * G1 + j) * C
            for c in range(4):
                w = y2l[pl.ds(rbase + c * 16, 16)]
                cr = w * s1[c]
                if j == 0:
                    s1a[c], sqa[c], cra[c] = s1[c], sq[c], cr
                else:
                    s1a[c] = s1a[c] + s1[c]
                    sqa[c] = sqa[c] + sq[c]
                    cra[c] = cra[c] + cr
        for c in range(4):
            acc[pl.ds(c * 16, 16)] = acc[pl.ds(c * 16, 16)] + s1a[c]
            acc[pl.ds(64 + c * 16, 16)] = acc[pl.ds(64 + c * 16, 16)] + sqa[c]
            acc[pl.ds(128 + c * 16, 16)] = acc[pl.ds(128 + c * 16, 16)] + cra[c]

    start(0, buf0, sem0)

    def body(g, carry):
        even = (g % 2) == 0
        more = g + 1 < NB1

        @pl.when(jnp.logical_and(even, more))
        def _():
            start(g + 1, buf1, sem1)

        @pl.when(jnp.logical_and(jnp.logical_not(even), more))
        def _():
            start(g + 1, buf0, sem0)

        @pl.when(even)
        def _():
            wait(g, buf0, sem0)
            compute(g, buf0)

        @pl.when(jnp.logical_not(even))
        def _():
            wait(g, buf1, sem1)
            compute(g, buf1)

        return carry

    lax.fori_loop(0, NB1, body, 0)
    pltpu.sync_copy(acc, part_hbm.at[pl.ds(wid * 192, 192)])


def _run_pass1(y1f, idx1, y2_1d):
    fn = pl.kernel(
        _pass1_body,
        out_type=jax.ShapeDtypeStruct((NW * 192,), jnp.float32),
        mesh=plsc.VectorSubcoreMesh(core_axis_name="c", subcore_axis_name="s"),
        compiler_params=pltpu.CompilerParams(use_tc_tiling_on_sc=False),
        scratch_types=[
            pltpu.VMEM((NB1, E1), jnp.int32),
            pltpu.VMEM((ROWS_W * C,), jnp.float32),
            pltpu.VMEM((E1, C), jnp.float32),
            pltpu.VMEM((E1, C), jnp.float32),
            pltpu.VMEM((192,), jnp.float32),
            pltpu.SemaphoreType.DMA,
            pltpu.SemaphoreType.DMA,
        ],
    )
    return fn(y1f, idx1, y2_1d)


def _stats_body(part_ref, y2_ref, gamma_ref, beta_ref, ab_ref):
    P = part_ref[...]
    sum1 = jnp.sum(P[:, 0, :], axis=0)
    ssq1 = jnp.sum(P[:, 1, :], axis=0)
    cross = jnp.sum(P[:, 2, :], axis=0)
    y2 = y2_ref[...]
    sy2 = jnp.sum(y2, axis=0)
    sy2q = jnp.sum(y2 * y2, axis=0)
    m = float(BN * KNB)
    s = sum1 + KNB * sy2
    q = ssq1 + 2.0 * cross + KNB * sy2q
    mean = s / m
    var = q / m - mean * mean
    a = gamma_ref[0] * lax.rsqrt(var + EPS)
    b2 = beta_ref[0] - mean * a
    ab_ref[0] = a
    ab_ref[1] = b2


def _run_stats(part3, y2f, gamma2, beta2):
    return pl.pallas_call(
        _stats_body,
        in_specs=[
            pl.BlockSpec((NW, 3, C), lambda: (0, 0, 0)),
            pl.BlockSpec((BN, C), lambda: (0, 0)),
            pl.BlockSpec((1, C), lambda: (0, 0)),
            pl.BlockSpec((1, C), lambda: (0, 0)),
        ],
        out_specs=pl.BlockSpec((2, C), lambda: (0, 0)),
        out_shape=jax.ShapeDtypeStruct((2, C), jnp.float32),
    )(part3, y2f, gamma2, beta2)


G2 = 4
NB2 = ROWS_W // G2
E2 = G2 * KNB


def _pass2_body(y1_hbm, idx_hbm, y2_hbm, ab_hbm, out_hbm,
                idxl, y2l, abl, buf0, buf1, obuf, sem0, sem1):
    wid = lax.axis_index("s") * NC + lax.axis_index("c")

    pltpu.sync_copy(idx_hbm.at[pl.ds(wid * NB2, NB2)], idxl)
    pltpu.sync_copy(y2_hbm.at[pl.ds(wid * ROWS_W * C, ROWS_W * C)], y2l)
    pltpu.sync_copy(ab_hbm, abl)

    av = [abl[pl.ds(c * 16, 16)] for c in range(4)]
    bv = [abl[pl.ds(64 + c * 16, 16)] for c in range(4)]

    def start(g, buf, sem):
        pltpu.async_copy(y1_hbm.at[idxl.at[g]], buf, sem)

    def wait(g, buf, sem):
        pltpu.make_async_copy(y1_hbm.at[idxl.at[g]], buf, sem).wait()

    def compute(g, buf):
        for j in range(G2):
            rbase = (g * G2 + j) * C
            zb = [None] * 4
            for c in range(4):
                zb[c] = av[c] * y2l[pl.ds(rbase + c * 16, 16)] + bv[c]
            o = [None] * 4
            for t in range(KNB):
                e = j * KNB + t
                for c in range(4):
                    u = av[c] * buf[e, pl.ds(c * 16, 16)] + zb[c]
                    r = jnp.maximum(u, ALPHA * u)
                    o[c] = r if t == 0 else o[c] + r
            for c in range(4):
                obuf[pl.ds(j * C + c * 16, 16)] = o[c] * (1.0 / KNB)
        base = (wid * ROWS_W + g * G2) * C
        pltpu.sync_copy(obuf, out_hbm.at[pl.ds(base, G2 * C)])

    start(0, buf0, sem0)

    def body(g, carry):
        even = (g % 2) == 0
        more = g + 1 < NB2

        @pl.when(jnp.logical_and(even, more))
        def _():
            start(g + 1, buf1, sem1)

        @pl.when(jnp.logical_and(jnp.logical_not(even), more))
        def _():
            start(g + 1, buf0, sem0)

        @pl.when(even)
        def _():
            wait(g, buf0, sem0)
            compute(g, buf0)

        @pl.when(jnp.logical_not(even))
        def _():
            wait(g, buf1, sem1)
            compute(g, buf1)

        return carry

    lax.fori_loop(0, NB2, body, 0)


def _run_pass2(y1f, idx2, y2_1d, ab1d):
    fn = pl.kernel(
        _pass2_body,
        out_type=jax.ShapeDtypeStruct((BN * C,), jnp.float32),
        mesh=plsc.VectorSubcoreMesh(core_axis_name="c", subcore_axis_name="s"),
        compiler_params=pltpu.CompilerParams(use_tc_tiling_on_sc=False),
        scratch_types=[
            pltpu.VMEM((NB2, E2), jnp.int32),
            pltpu.VMEM((ROWS_W * C,), jnp.float32),
            pltpu.VMEM((2 * C,), jnp.float32),
            pltpu.VMEM((E2, C), jnp.float32),
            pltpu.VMEM((E2, C), jnp.float32),
            pltpu.VMEM((G2 * C,), jnp.float32),
            pltpu.SemaphoreType.DMA,
            pltpu.SemaphoreType.DMA,
        ],
    )
    return fn(y1f, idx2, y2_1d, ab1d)


def kernel(points, x, W, gamma, beta):
    xt = jnp.transpose(x, (0, 2, 1))
    w1t = jnp.transpose(W[:, :D])
    w2t = jnp.transpose(W[:, D:] - W[:, :D])

    idx, y1, y2 = _run_knn(points, xt, w1t, w2t)

    y1f = y1.reshape(BN, C)
    y2f = y2.reshape(BN, C)
    y2_1d = y2.reshape(BN * C)
    idx1 = idx.reshape(NW * NB1, E1)
    idx2 = idx.reshape(NW * NB2, E2)

    part = _run_pass1(y1f, idx1, y2_1d)
    ab = _run_stats(part.reshape(NW, 3, C), y2f,
                    gamma.reshape(1, C), beta.reshape(1, C))
    out_t = _run_pass2(y1f, idx2, y2_1d, ab.reshape(2 * C))
    return jnp.transpose(out_t.reshape(B, N, C), (0, 2, 1))
